# Initial kernel scaffold; baseline (speedup 1.0000x reference)
import jax
import jax.numpy as jnp
from jax.experimental import pallas as pl


def kernel(input_ids, target_ids, emb_in, emb_tgt, W_ih_e, W_hh_e, b_ih_e, b_hh_e, W_ih_d, W_hh_d, b_ih_d, b_hh_d, W_tl, b_tl, W_lin, b_lin):
    raise NotImplementedError("write your pallas kernel here")


# speedup vs baseline: 6.3908x; 6.3908x over previous
import functools

import jax
import jax.numpy as jnp
from jax import lax
from jax.experimental import pallas as pl
from jax.experimental.pallas import tpu as pltpu
from jax.experimental.pallas import tpu_sc as plsc


def _dot_t(a, w):
    return lax.dot_general(a, w, (((1,), (1,)), ((), ())),
                           preferred_element_type=jnp.float32)


def _sc_gather_pair(emb_a, idx_a, emb_b, idx_b):
    n = idx_a.shape[0]
    h = emb_a.shape[1]
    info = plsc.get_sparse_core_info()
    nw = info.num_cores * info.num_subcores
    n_per = n // nw

    @functools.partial(
        pl.kernel,
        out_type=(jax.ShapeDtypeStruct((n, h), jnp.float32),
                  jax.ShapeDtypeStruct((n, h), jnp.float32)),
        mesh=plsc.VectorSubcoreMesh(core_axis_name="c", subcore_axis_name="s"),
        scratch_types=[
            pltpu.VMEM((n_per,), jnp.int32),
            pltpu.VMEM((n_per, h), jnp.float32),
            pltpu.SemaphoreType.DMA,
        ],
    )
    def k(emb_a_hbm, idx_a_hbm, emb_b_hbm, idx_b_hbm, out_a, out_b,
          idx_v, rows_v, sem):
        wid = lax.axis_index("s") * info.num_cores + lax.axis_index("c")
        base = wid * n_per
        pltpu.sync_copy(idx_a_hbm.at[pl.ds(base, n_per)], idx_v)
        pltpu.async_copy(emb_a_hbm.at[idx_v], rows_v, sem).wait()
        pltpu.sync_copy(rows_v, out_a.at[pl.ds(base, n_per)])
        pltpu.sync_copy(idx_b_hbm.at[pl.ds(base, n_per)], idx_v)
        pltpu.async_copy(emb_b_hbm.at[idx_v], rows_v, sem).wait()
        pltpu.sync_copy(rows_v, out_b.at[pl.ds(base, n_per)])

    return k(emb_a, idx_a, emb_b, idx_b)


def _lstm_gates(x, h, w_ih, w_hh, bias):
    gates = _dot_t(x, w_ih) + _dot_t(h, w_hh) + bias
    hh = h.shape[1]
    i = jax.nn.sigmoid(gates[:, :hh])
    f = jax.nn.sigmoid(gates[:, hh:2 * hh])
    g = jnp.tanh(gates[:, 2 * hh:3 * hh])
    o = jax.nn.sigmoid(gates[:, 3 * hh:])
    return i, f, g, o


def _encoder(x_seq, ids3, w_ih, w_hh, bias):
    s_len, b, h = x_seq.shape

    def body(x_ref, ids_ref, wih_ref, whh_ref, b_ref, h_out, c_out,
             h_scr, c_scr):
        s = pl.program_id(0)

        @pl.when(s == 0)
        def _():
            h_scr[...] = jnp.zeros_like(h_scr)
            c_scr[...] = jnp.zeros_like(c_scr)

        mask = (ids_ref[0, 0, :] != 0).astype(jnp.float32)
        x = x_ref[0] * mask[:, None]
        hprev = h_scr[...]
        c = c_scr[...]
        i, f, g, o = _lstm_gates(x, hprev, wih_ref[...], whh_ref[...],
                                 b_ref[...])
        c2 = f * c + i * g
        h2 = o * jnp.tanh(c2)
        h_scr[...] = h2
        c_scr[...] = c2

        @pl.when(s == s_len - 1)
        def _():
            h_out[...] = h2
            c_out[...] = c2

    return pl.pallas_call(
        body,
        grid=(s_len,),
        in_specs=[
            pl.BlockSpec((1, b, h), lambda s: (s, 0, 0)),
            pl.BlockSpec((1, 1, b), lambda s: (s, 0, 0)),
            pl.BlockSpec(w_ih.shape, lambda s: (0, 0)),
            pl.BlockSpec(w_hh.shape, lambda s: (0, 0)),
            pl.BlockSpec(bias.shape, lambda s: (0, 0)),
        ],
        out_specs=[
            pl.BlockSpec((b, h), lambda s: (0, 0)),
            pl.BlockSpec((b, h), lambda s: (0, 0)),
        ],
        out_shape=[
            jax.ShapeDtypeStruct((b, h), jnp.float32),
            jax.ShapeDtypeStruct((b, h), jnp.float32),
        ],
        scratch_shapes=[
            pltpu.VMEM((b, h), jnp.float32),
            pltpu.VMEM((b, h), jnp.float32),
        ],
        compiler_params=pltpu.CompilerParams(
            dimension_semantics=("arbitrary",)),
    )(x_seq, ids3, w_ih, w_hh, bias)


def _decoder(x_seq, ids3, w_ih, w_hh, bias, h_enc, c_enc, w_tl_h, w_tl_e,
             b_tl):
    s_len, b, h = x_seq.shape

    def body(x_ref, ids_ref, wih_ref, whh_ref, b_ref, he_ref, ce_ref,
             wtlh_ref, wtle_ref, btl_ref, comb_out, h_scr, c_scr, const_scr):
        s = pl.program_id(0)

        @pl.when(s == 0)
        def _():
            h_scr[...] = he_ref[...]
            c_scr[...] = ce_ref[...]
            const_scr[...] = _dot_t(he_ref[...], wtle_ref[...]) + btl_ref[...]

        mask = (ids_ref[0, 0, :] != 0).astype(jnp.float32)
        x = x_ref[0] * mask[:, None]
        hprev = h_scr[...]
        c = c_scr[...]
        i, f, g, o = _lstm_gates(x, hprev, wih_ref[...], whh_ref[...],
                                 b_ref[...])
        c2 = f * c + i * g
        h2 = o * jnp.tanh(c2)
        h_scr[...] = h2
        c_scr[...] = c2
        comb_out[0] = jnp.tanh(_dot_t(h2, wtlh_ref[...]) + const_scr[...])

    return pl.pallas_call(
        body,
        grid=(s_len,),
        in_specs=[
            pl.BlockSpec((1, b, h), lambda s: (s, 0, 0)),
            pl.BlockSpec((1, 1, b), lambda s: (s, 0, 0)),
            pl.BlockSpec(w_ih.shape, lambda s: (0, 0)),
            pl.BlockSpec(w_hh.shape, lambda s: (0, 0)),
            pl.BlockSpec(bias.shape, lambda s: (0, 0)),
            pl.BlockSpec((b, h), lambda s: (0, 0)),
            pl.BlockSpec((b, h), lambda s: (0, 0)),
            pl.BlockSpec(w_tl_h.shape, lambda s: (0, 0)),
            pl.BlockSpec(w_tl_e.shape, lambda s: (0, 0)),
            pl.BlockSpec(b_tl.shape, lambda s: (0, 0)),
        ],
        out_specs=pl.BlockSpec((1, b, h), lambda s: (s, 0, 0)),
        out_shape=jax.ShapeDtypeStruct((s_len, b, h), jnp.float32),
        scratch_shapes=[
            pltpu.VMEM((b, h), jnp.float32),
            pltpu.VMEM((b, h), jnp.float32),
            pltpu.VMEM((b, h), jnp.float32),
        ],
        compiler_params=pltpu.CompilerParams(
            dimension_semantics=("arbitrary",)),
    )(x_seq, ids3, w_ih, w_hh, bias, h_enc, c_enc, w_tl_h, w_tl_e, b_tl)


def _project(comb, w_lin, b_lin):
    m, h = comb.shape
    v = w_lin.shape[0]
    bm, bn = 512, 2048

    def body(c_ref, w_ref, b_ref, o_ref):
        o_ref[...] = _dot_t(c_ref[...], w_ref[...]) + b_ref[...]

    return pl.pallas_call(
        body,
        grid=(v // bn, m // bm),
        in_specs=[
            pl.BlockSpec((bm, h), lambda n, mm: (mm, 0)),
            pl.BlockSpec((bn, h), lambda n, mm: (n, 0)),
            pl.BlockSpec((1, bn), lambda n, mm: (0, n)),
        ],
        out_specs=pl.BlockSpec((bm, bn), lambda n, mm: (mm, n)),
        out_shape=jax.ShapeDtypeStruct((m, v), jnp.float32),
        compiler_params=pltpu.CompilerParams(
            dimension_semantics=("arbitrary", "arbitrary")),
    )(comb, w_lin, b_lin)


def kernel(input_ids, target_ids, emb_in, emb_tgt, W_ih_e, W_hh_e, b_ih_e,
           b_hh_e, W_ih_d, W_hh_d, b_ih_d, b_hh_d, W_tl, b_tl, W_lin, b_lin):
    b, s_in = input_ids.shape
    s_out = target_ids.shape[1]
    h = W_hh_e.shape[1]
    v = W_lin.shape[0]

    ids_in = input_ids.T.reshape(-1)
    ids_tgt = target_ids.T.reshape(-1)
    x_in_flat, x_tgt_flat = _sc_gather_pair(emb_in, ids_in, emb_tgt, ids_tgt)
    x_in = x_in_flat.reshape(s_in, b, h)
    x_tgt = x_tgt_flat.reshape(s_out, b, h)

    h_enc, c_enc = _encoder(x_in, ids_in.reshape(s_in, 1, b),
                            W_ih_e, W_hh_e, (b_ih_e + b_hh_e).reshape(1, -1))
    comb = _decoder(x_tgt, ids_tgt.reshape(s_out, 1, b),
                    W_ih_d, W_hh_d, (b_ih_d + b_hh_d).reshape(1, -1),
                    h_enc, c_enc, W_tl[:, :h], W_tl[:, h:],
                    b_tl.reshape(1, -1))
    comb_flat = comb.transpose(1, 0, 2).reshape(b * s_out, h)
    logits = _project(comb_flat, W_lin, b_lin.reshape(1, -1))
    return logits.reshape(b, s_out, v)


# speedup vs baseline: 8.1903x; 1.2816x over previous
import functools

import jax
import jax.numpy as jnp
from jax import lax
from jax.experimental import pallas as pl
from jax.experimental.pallas import tpu as pltpu
from jax.experimental.pallas import tpu_sc as plsc


def _dot_t(a, w):
    return lax.dot_general(a, w, (((1,), (1,)), ((), ())),
                           preferred_element_type=jnp.float32)


def _sc_gather_pair(emb_a, idx_a, emb_b, idx_b):
    n = idx_a.shape[0]
    h = emb_a.shape[1]
    info = plsc.get_sparse_core_info()
    nw = info.num_cores * info.num_subcores
    n_per = n // nw

    @functools.partial(
        pl.kernel,
        out_type=(jax.ShapeDtypeStruct((n, h), jnp.float32),
                  jax.ShapeDtypeStruct((n, h), jnp.float32)),
        mesh=plsc.VectorSubcoreMesh(core_axis_name="c", subcore_axis_name="s"),
        scratch_types=[
            pltpu.VMEM((n_per,), jnp.int32),
            pltpu.VMEM((n_per, h), jnp.float32),
            pltpu.SemaphoreType.DMA,
        ],
    )
    def k(emb_a_hbm, idx_a_hbm, emb_b_hbm, idx_b_hbm, out_a, out_b,
          idx_v, rows_v, sem):
        wid = lax.axis_index("s") * info.num_cores + lax.axis_index("c")
        base = wid * n_per
        pltpu.sync_copy(idx_a_hbm.at[pl.ds(base, n_per)], idx_v)
        pltpu.async_copy(emb_a_hbm.at[idx_v], rows_v, sem).wait()
        pltpu.sync_copy(rows_v, out_a.at[pl.ds(base, n_per)])
        pltpu.sync_copy(idx_b_hbm.at[pl.ds(base, n_per)], idx_v)
        pltpu.async_copy(emb_b_hbm.at[idx_v], rows_v, sem).wait()
        pltpu.sync_copy(rows_v, out_b.at[pl.ds(base, n_per)])

    return k(emb_a, idx_a, emb_b, idx_b)


def _split_gates(gates, hh):
    i = jax.nn.sigmoid(gates[:, :hh])
    f = jax.nn.sigmoid(gates[:, hh:2 * hh])
    g = jnp.tanh(gates[:, 2 * hh:3 * hh])
    o = jax.nn.sigmoid(gates[:, 3 * hh:])
    return i, f, g, o


def _masked_xw(x, ids3, w, bias):
    m, h = x.shape
    n4 = w.shape[0]
    bm, bn = 512, 2048

    def body(x_ref, ids_ref, w_ref, b_ref, o_ref):
        mask = (ids_ref[0, 0, :] != 0).astype(jnp.float32)
        o_ref[...] = _dot_t(x_ref[...] * mask[:, None], w_ref[...]) + b_ref[...]

    return pl.pallas_call(
        body,
        grid=(n4 // bn, m // bm),
        in_specs=[
            pl.BlockSpec((bm, h), lambda n, mm: (mm, 0)),
            pl.BlockSpec((1, 1, bm), lambda n, mm: (mm, 0, 0)),
            pl.BlockSpec((bn, h), lambda n, mm: (n, 0)),
            pl.BlockSpec((1, bn), lambda n, mm: (0, n)),
        ],
        out_specs=pl.BlockSpec((bm, bn), lambda n, mm: (mm, n)),
        out_shape=jax.ShapeDtypeStruct((m, n4), jnp.float32),
        compiler_params=pltpu.CompilerParams(
            dimension_semantics=("arbitrary", "arbitrary")),
    )(x, ids3, w, bias)


def _encoder(xw_seq, w_hh):
    s_len, b, h4 = xw_seq.shape
    h = h4 // 4

    def body(xw_ref, whh_ref, h_out, c_out, h_scr, c_scr):
        s = pl.program_id(0)

        @pl.when(s == 0)
        def _():
            h_scr[...] = jnp.zeros_like(h_scr)
            c_scr[...] = jnp.zeros_like(c_scr)

        hprev = h_scr[...]
        c = c_scr[...]
        gates = xw_ref[0] + _dot_t(hprev, whh_ref[...])
        i, f, g, o = _split_gates(gates, h)
        c2 = f * c + i * g
        h2 = o * jnp.tanh(c2)
        h_scr[...] = h2
        c_scr[...] = c2

        @pl.when(s == s_len - 1)
        def _():
            h_out[...] = h2
            c_out[...] = c2

    return pl.pallas_call(
        body,
        grid=(s_len,),
        in_specs=[
            pl.BlockSpec((1, b, h4), lambda s: (s, 0, 0)),
            pl.BlockSpec(w_hh.shape, lambda s: (0, 0)),
        ],
        out_specs=[
            pl.BlockSpec((b, h), lambda s: (0, 0)),
            pl.BlockSpec((b, h), lambda s: (0, 0)),
        ],
        out_shape=[
            jax.ShapeDtypeStruct((b, h), jnp.float32),
            jax.ShapeDtypeStruct((b, h), jnp.float32),
        ],
        scratch_shapes=[
            pltpu.VMEM((b, h), jnp.float32),
            pltpu.VMEM((b, h), jnp.float32),
        ],
        compiler_params=pltpu.CompilerParams(
            dimension_semantics=("arbitrary",)),
    )(xw_seq, w_hh)


def _decoder(xw_seq, w_hh, h_enc, c_enc, w_tl_h, w_tl_e, b_tl):
    s_len, b, h4 = xw_seq.shape
    h = h4 // 4

    def body(xw_ref, whh_ref, he_ref, ce_ref, wtlh_ref, wtle_ref, btl_ref,
             comb_out, h_scr, c_scr, const_scr):
        s = pl.program_id(0)

        @pl.when(s == 0)
        def _():
            h_scr[...] = he_ref[...]
            c_scr[...] = ce_ref[...]
            const_scr[...] = _dot_t(he_ref[...], wtle_ref[...]) + btl_ref[...]

        hprev = h_scr[...]
        c = c_scr[...]
        gates = xw_ref[0] + _dot_t(hprev, whh_ref[...])
        i, f, g, o = _split_gates(gates, h)
        c2 = f * c + i * g
        h2 = o * jnp.tanh(c2)
        h_scr[...] = h2
        c_scr[...] = c2
        comb_out[0] = jnp.tanh(_dot_t(h2, wtlh_ref[...]) + const_scr[...])

    return pl.pallas_call(
        body,
        grid=(s_len,),
        in_specs=[
            pl.BlockSpec((1, b, h4), lambda s: (s, 0, 0)),
            pl.BlockSpec(w_hh.shape, lambda s: (0, 0)),
            pl.BlockSpec((b, h), lambda s: (0, 0)),
            pl.BlockSpec((b, h), lambda s: (0, 0)),
            pl.BlockSpec(w_tl_h.shape, lambda s: (0, 0)),
            pl.BlockSpec(w_tl_e.shape, lambda s: (0, 0)),
            pl.BlockSpec(b_tl.shape, lambda s: (0, 0)),
        ],
        out_specs=pl.BlockSpec((1, b, h), lambda s: (s, 0, 0)),
        out_shape=jax.ShapeDtypeStruct((s_len, b, h), jnp.float32),
        scratch_shapes=[
            pltpu.VMEM((b, h), jnp.float32),
            pltpu.VMEM((b, h), jnp.float32),
            pltpu.VMEM((b, h), jnp.float32),
        ],
        compiler_params=pltpu.CompilerParams(
            dimension_semantics=("arbitrary",)),
    )(xw_seq, w_hh, h_enc, c_enc, w_tl_h, w_tl_e, b_tl)


def _project(comb, w_lin, b_lin):
    m, h = comb.shape
    v = w_lin.shape[0]
    bm, bn = 512, 2048

    def body(c_ref, w_ref, b_ref, o_ref):
        o_ref[...] = _dot_t(c_ref[...], w_ref[...]) + b_ref[...]

    return pl.pallas_call(
        body,
        grid=(v // bn, m // bm),
        in_specs=[
            pl.BlockSpec((bm, h), lambda n, mm: (mm, 0)),
            pl.BlockSpec((bn, h), lambda n, mm: (n, 0)),
            pl.BlockSpec((1, bn), lambda n, mm: (0, n)),
        ],
        out_specs=pl.BlockSpec((bm, bn), lambda n, mm: (mm, n)),
        out_shape=jax.ShapeDtypeStruct((m, v), jnp.float32),
        compiler_params=pltpu.CompilerParams(
            dimension_semantics=("arbitrary", "arbitrary")),
    )(comb, w_lin, b_lin)


def kernel(input_ids, target_ids, emb_in, emb_tgt, W_ih_e, W_hh_e, b_ih_e,
           b_hh_e, W_ih_d, W_hh_d, b_ih_d, b_hh_d, W_tl, b_tl, W_lin, b_lin):
    b, s_in = input_ids.shape
    s_out = target_ids.shape[1]
    h = W_hh_e.shape[1]
    v = W_lin.shape[0]

    ids_in = input_ids.T.reshape(-1)
    ids_tgt = target_ids.T.reshape(-1)
    x_in_flat, x_tgt_flat = _sc_gather_pair(emb_in, ids_in, emb_tgt, ids_tgt)

    bm = 512
    xw_in = _masked_xw(x_in_flat, ids_in.reshape(s_in * b // bm, 1, bm),
                       W_ih_e, (b_ih_e + b_hh_e).reshape(1, -1))
    xw_tgt = _masked_xw(x_tgt_flat, ids_tgt.reshape(s_out * b // bm, 1, bm),
                        W_ih_d, (b_ih_d + b_hh_d).reshape(1, -1))
    h_enc, c_enc = _encoder(xw_in.reshape(s_in, b, 4 * h), W_hh_e)
    comb = _decoder(xw_tgt.reshape(s_out, b, 4 * h), W_hh_d,
                    h_enc, c_enc, W_tl[:, :h], W_tl[:, h:],
                    b_tl.reshape(1, -1))
    comb_flat = comb.transpose(1, 0, 2).reshape(b * s_out, h)
    logits = _project(comb_flat, W_lin, b_lin.reshape(1, -1))
    return logits.reshape(b, s_out, v)
